# baseline (speedup 1.0000x reference)
import jax
import jax.numpy as jnp
from jax import lax
from jax.experimental import pallas as pl
from jax.experimental.pallas import tpu as pltpu
from jax.experimental.pallas import tpu_sc as plsc

_N_CLASSES = 100000
_SMOOTHING = 0.1
_CONFIDENCE = 1.0 - _SMOOTHING
_EPS = _SMOOTHING / (_N_CLASSES - 1)

_ROWS = 1024

_C_SC = 45056
_TC_BR = 4096
_TC_BLK0 = _C_SC // _TC_BR
_TC_NBLK = (_N_CLASSES - _C_SC + _TC_BR - 1) // _TC_BR

_NC = 2
_NS = 16
_NW = _NC * _NS
_CPW = _C_SC // _NW
_SLAB = 32
_NSLAB = _CPW // _SLAB
_RG = _ROWS // 16


def _tc_kernel(lsmt_ref, tgt_ref, out_ref):
    j = pl.program_id(0)
    blk = lsmt_ref[...]
    cls = jax.lax.broadcasted_iota(jnp.int32, (_TC_BR, _ROWS), 0) + (
        _C_SC + j * _TC_BR
    )
    blk0 = jnp.where(cls < _N_CLASSES, blk, 0.0)
    s = jnp.sum(blk0)
    tgt = tgt_ref[...]
    g = jnp.sum(jnp.where(cls == tgt, blk, 0.0))
    out_ref[...] = jnp.reshape(_EPS * s + (_CONFIDENCE - _EPS) * g, (1, 1, 1))


def _tc_call(lsmt, tgt2d):
    return pl.pallas_call(
        _tc_kernel,
        grid=(_TC_NBLK,),
        in_specs=[
            pl.BlockSpec((_TC_BR, _ROWS), lambda j: (_TC_BLK0 + j, 0)),
            pl.BlockSpec((1, _ROWS), lambda j: (0, 0)),
        ],
        out_specs=pl.BlockSpec((1, 1, 1), lambda j: (j, 0, 0)),
        out_shape=jax.ShapeDtypeStruct((_TC_NBLK, 1, 1), jnp.float32),
        compiler_params=pltpu.CompilerParams(
            dimension_semantics=("arbitrary",),
        ),
    )(lsmt, tgt2d)


def _process_slab(buf, tvbuf, c0, carry):
    def rg_body(rg, carry):
        acc_s, acc_g = carry
        t_slice = tvbuf[pl.ds(rg * 16, 16)]
        for col in range(_SLAB):
            v = buf[col, pl.ds(rg * 16, 16)]
            acc_s = acc_s + v
            acc_g = acc_g + jnp.where(t_slice == c0 + col, v, 0.0)
        return acc_s, acc_g
    return lax.fori_loop(0, _RG, rg_body, carry)


def _sc_body(lsmt_hbm, tgt_hbm, out_hbm, tvbuf, buf0, buf1, outbuf, sem0, sem1):
    c = lax.axis_index("c")
    s = lax.axis_index("s")
    wid = s * _NC + c
    cbase = wid * _CPW
    pltpu.sync_copy(tgt_hbm, tvbuf)

    def slab_src(k):
        c0 = pl.multiple_of(cbase + k * _SLAB, 8)
        return lsmt_hbm.at[pl.ds(c0, _SLAB), :], c0

    src0, _ = slab_src(0)
    src1, _ = slab_src(1)
    pltpu.make_async_copy(src0, buf0, sem0).start()
    pltpu.make_async_copy(src1, buf1, sem1).start()

    def pair_step(p, carry):
        k = p * 2

        src_a, c0_a = slab_src(k)
        pltpu.make_async_copy(src_a, buf0, sem0).wait()
        carry = _process_slab(buf0, tvbuf, c0_a, carry)

        @pl.when(k + 2 < _NSLAB)
        def _():
            src_n, _ = slab_src(k + 2)
            pltpu.make_async_copy(src_n, buf0, sem0).start()

        src_b, c0_b = slab_src(k + 1)
        pltpu.make_async_copy(src_b, buf1, sem1).wait()
        carry = _process_slab(buf1, tvbuf, c0_b, carry)

        @pl.when(k + 3 < _NSLAB)
        def _():
            src_n, _ = slab_src(k + 3)
            pltpu.make_async_copy(src_n, buf1, sem1).start()

        return carry

    zero = jnp.zeros((16,), jnp.float32)
    acc_s, acc_g = lax.fori_loop(0, _NSLAB // 2, pair_step, (zero, zero))
    outbuf[...] = _EPS * acc_s + (_CONFIDENCE - _EPS) * acc_g
    pltpu.sync_copy(outbuf, out_hbm.at[pl.ds(wid * 16, 16)])


_sc_call = pl.kernel(
    _sc_body,
    out_type=jax.ShapeDtypeStruct((_NW * 16,), jnp.float32),
    mesh=plsc.VectorSubcoreMesh(
        core_axis_name="c", subcore_axis_name="s", num_cores=_NC, num_subcores=_NS
    ),
    scratch_types=[
        pltpu.VMEM((_ROWS,), jnp.int32),
        pltpu.VMEM((_SLAB, _ROWS), jnp.float32),
        pltpu.VMEM((_SLAB, _ROWS), jnp.float32),
        pltpu.VMEM((16,), jnp.float32),
        pltpu.SemaphoreType.DMA,
        pltpu.SemaphoreType.DMA,
    ],
)


def kernel(lsm, target):
    lsmt = lsm.T
    tgt = target.astype(jnp.int32)
    sc_partials = _sc_call(lsmt, tgt)
    tc_partials = _tc_call(lsmt, tgt.reshape(1, _ROWS))
    return -(jnp.sum(sc_partials) + jnp.sum(tc_partials)) / _ROWS


# speedup vs baseline: 1.0233x; 1.0233x over previous
import jax
import jax.numpy as jnp
from jax import lax
from jax.experimental import pallas as pl
from jax.experimental.pallas import tpu as pltpu
from jax.experimental.pallas import tpu_sc as plsc

_N_CLASSES = 100000
_SMOOTHING = 0.1
_CONFIDENCE = 1.0 - _SMOOTHING
_EPS = _SMOOTHING / (_N_CLASSES - 1)

_ROWS = 1024

_C_SC = 47104
_TC_BR = 2048
_TC_BLK0 = _C_SC // _TC_BR
_TC_NBLK = (_N_CLASSES - _C_SC + _TC_BR - 1) // _TC_BR

_NC = 2
_NS = 16
_NW = _NC * _NS
_CPW = _C_SC // _NW
_SLAB = 32
_NSLAB = _CPW // _SLAB
_RG = _ROWS // 16


def _tc_kernel(lsmt_ref, tgt_ref, out_ref):
    j = pl.program_id(0)
    blk = lsmt_ref[...]
    cls = jax.lax.broadcasted_iota(jnp.int32, (_TC_BR, _ROWS), 0) + (
        _C_SC + j * _TC_BR
    )
    blk0 = jnp.where(cls < _N_CLASSES, blk, 0.0)
    s = jnp.sum(blk0)
    tgt = tgt_ref[...]
    g = jnp.sum(jnp.where(cls == tgt, blk, 0.0))
    out_ref[...] = jnp.reshape(_EPS * s + (_CONFIDENCE - _EPS) * g, (1, 1, 1))


def _tc_call(lsmt, tgt2d):
    return pl.pallas_call(
        _tc_kernel,
        grid=(_TC_NBLK,),
        in_specs=[
            pl.BlockSpec((_TC_BR, _ROWS), lambda j: (_TC_BLK0 + j, 0)),
            pl.BlockSpec((1, _ROWS), lambda j: (0, 0)),
        ],
        out_specs=pl.BlockSpec((1, 1, 1), lambda j: (j, 0, 0)),
        out_shape=jax.ShapeDtypeStruct((_TC_NBLK, 1, 1), jnp.float32),
        compiler_params=pltpu.CompilerParams(
            dimension_semantics=("arbitrary",),
        ),
    )(lsmt, tgt2d)


def _process_slab(buf, tvbuf, c0, carry):
    def rg_body(rg, carry):
        acc_s, acc_g = carry
        t_slice = tvbuf[pl.ds(rg * 16, 16)]
        for col in range(_SLAB):
            v = buf[col, pl.ds(rg * 16, 16)]
            acc_s = acc_s + v
            acc_g = acc_g + jnp.where(t_slice == c0 + col, v, 0.0)
        return acc_s, acc_g
    return lax.fori_loop(0, _RG, rg_body, carry)


def _sc_body(lsmt_hbm, tgt_hbm, out_hbm, tvbuf, buf0, buf1, outbuf, sem0, sem1):
    c = lax.axis_index("c")
    s = lax.axis_index("s")
    wid = s * _NC + c
    cbase = wid * _CPW
    pltpu.sync_copy(tgt_hbm, tvbuf)

    half = _SLAB // 2

    def start_slab(k, buf, sem):
        c0 = pl.multiple_of(cbase + k * _SLAB, 8)
        pltpu.make_async_copy(
            lsmt_hbm.at[pl.ds(c0, half), :], buf.at[pl.ds(0, half), :], sem
        ).start()
        pltpu.make_async_copy(
            lsmt_hbm.at[pl.ds(c0 + half, half), :],
            buf.at[pl.ds(half, half), :],
            sem,
        ).start()

    def wait_slab(k, buf, sem):
        c0 = pl.multiple_of(cbase + k * _SLAB, 8)
        pltpu.make_async_copy(
            lsmt_hbm.at[pl.ds(c0, _SLAB), :], buf, sem
        ).wait()
        return c0

    start_slab(0, buf0, sem0)
    start_slab(1, buf1, sem1)

    def pair_step(p, carry):
        k = p * 2

        c0_a = wait_slab(k, buf0, sem0)
        carry = _process_slab(buf0, tvbuf, c0_a, carry)

        @pl.when(k + 2 < _NSLAB)
        def _():
            start_slab(k + 2, buf0, sem0)

        c0_b = wait_slab(k + 1, buf1, sem1)
        carry = _process_slab(buf1, tvbuf, c0_b, carry)

        @pl.when(k + 3 < _NSLAB)
        def _():
            start_slab(k + 3, buf1, sem1)

        return carry

    zero = jnp.zeros((16,), jnp.float32)
    acc_s, acc_g = lax.fori_loop(0, _NSLAB // 2, pair_step, (zero, zero))
    outbuf[...] = _EPS * acc_s + (_CONFIDENCE - _EPS) * acc_g
    pltpu.sync_copy(outbuf, out_hbm.at[pl.ds(wid * 16, 16)])


_sc_call = pl.kernel(
    _sc_body,
    out_type=jax.ShapeDtypeStruct((_NW * 16,), jnp.float32),
    mesh=plsc.VectorSubcoreMesh(
        core_axis_name="c", subcore_axis_name="s", num_cores=_NC, num_subcores=_NS
    ),
    scratch_types=[
        pltpu.VMEM((_ROWS,), jnp.int32),
        pltpu.VMEM((_SLAB, _ROWS), jnp.float32),
        pltpu.VMEM((_SLAB, _ROWS), jnp.float32),
        pltpu.VMEM((16,), jnp.float32),
        pltpu.SemaphoreType.DMA,
        pltpu.SemaphoreType.DMA,
    ],
)


def kernel(lsm, target):
    lsmt = lsm.T
    tgt = target.astype(jnp.int32)
    sc_partials = _sc_call(lsmt, tgt)
    tc_partials = _tc_call(lsmt, tgt.reshape(1, _ROWS))
    return -(jnp.sum(sc_partials) + jnp.sum(tc_partials)) / _ROWS


# speedup vs baseline: 1.0284x; 1.0050x over previous
import jax
import jax.numpy as jnp
from jax import lax
from jax.experimental import pallas as pl
from jax.experimental.pallas import tpu as pltpu
from jax.experimental.pallas import tpu_sc as plsc

_N_CLASSES = 100000
_SMOOTHING = 0.1
_CONFIDENCE = 1.0 - _SMOOTHING
_EPS = _SMOOTHING / (_N_CLASSES - 1)

_ROWS = 1024

_C_SC = 47104
_TC_BR = 2048
_TC_BLK0 = _C_SC // _TC_BR
_TC_NBLK = (_N_CLASSES - _C_SC + _TC_BR - 1) // _TC_BR

_NC = 2
_NS = 16
_NW = _NC * _NS
_CPW = _C_SC // _NW
_SLAB = 32
_NSLAB = _CPW // _SLAB
_RG = _ROWS // 16


def _tc_kernel(lsmt_ref, tgt_ref, out_ref):
    j = pl.program_id(0)
    blk = lsmt_ref[...]
    cls = jax.lax.broadcasted_iota(jnp.int32, (_TC_BR, _ROWS), 0) + (
        _C_SC + j * _TC_BR
    )
    blk0 = jnp.where(cls < _N_CLASSES, blk, 0.0)
    s = jnp.sum(blk0)
    tgt = tgt_ref[...]
    g = jnp.sum(jnp.where(cls == tgt, blk, 0.0))
    acc = jnp.reshape(_EPS * s + (_CONFIDENCE - _EPS) * g, (1, 1))

    @pl.when(j == 0)
    def _():
        out_ref[...] = jnp.zeros_like(out_ref)

    out_ref[...] += acc


def _tc_call(lsmt, tgt2d):
    return pl.pallas_call(
        _tc_kernel,
        grid=(_TC_NBLK,),
        in_specs=[
            pl.BlockSpec((_TC_BR, _ROWS), lambda j: (_TC_BLK0 + j, 0)),
            pl.BlockSpec((1, _ROWS), lambda j: (0, 0)),
        ],
        out_specs=pl.BlockSpec((1, 1), lambda j: (0, 0)),
        out_shape=jax.ShapeDtypeStruct((1, 1), jnp.float32),
        compiler_params=pltpu.CompilerParams(
            dimension_semantics=("arbitrary",),
        ),
    )(lsmt, tgt2d)


def _process_slab(buf, tvbuf, c0, carry):
    def rg_body(rg, carry):
        acc_s, acc_g = carry
        t_slice = tvbuf[pl.ds(rg * 16, 16)]
        for col in range(_SLAB):
            v = buf[col, pl.ds(rg * 16, 16)]
            acc_s = acc_s + v
            acc_g = acc_g + jnp.where(t_slice == c0 + col, v, 0.0)
        return acc_s, acc_g
    return lax.fori_loop(0, _RG, rg_body, carry)


def _sc_body(lsmt_hbm, tgt_hbm, out_hbm, tvbuf, buf0, buf1, outbuf, sem0, sem1):
    c = lax.axis_index("c")
    s = lax.axis_index("s")
    wid = s * _NC + c
    cbase = wid * _CPW
    pltpu.sync_copy(tgt_hbm, tvbuf)

    def slab_src(k):
        c0 = pl.multiple_of(cbase + k * _SLAB, 8)
        return lsmt_hbm.at[pl.ds(c0, _SLAB), :], c0

    src0, _ = slab_src(0)
    src1, _ = slab_src(1)
    pltpu.make_async_copy(src0, buf0, sem0).start()
    pltpu.make_async_copy(src1, buf1, sem1).start()

    def pair_step(p, carry):
        k = p * 2

        src_a, c0_a = slab_src(k)
        pltpu.make_async_copy(src_a, buf0, sem0).wait()
        carry = _process_slab(buf0, tvbuf, c0_a, carry)

        @pl.when(k + 2 < _NSLAB)
        def _():
            src_n, _ = slab_src(k + 2)
            pltpu.make_async_copy(src_n, buf0, sem0).start()

        src_b, c0_b = slab_src(k + 1)
        pltpu.make_async_copy(src_b, buf1, sem1).wait()
        carry = _process_slab(buf1, tvbuf, c0_b, carry)

        @pl.when(k + 3 < _NSLAB)
        def _():
            src_n, _ = slab_src(k + 3)
            pltpu.make_async_copy(src_n, buf1, sem1).start()

        return carry

    zero = jnp.zeros((16,), jnp.float32)
    acc_s, acc_g = lax.fori_loop(0, _NSLAB // 2, pair_step, (zero, zero))
    outbuf[...] = _EPS * acc_s + (_CONFIDENCE - _EPS) * acc_g
    pltpu.sync_copy(outbuf, out_hbm.at[pl.ds(wid * 16, 16)])


_sc_call = pl.kernel(
    _sc_body,
    out_type=jax.ShapeDtypeStruct((_NW * 16,), jnp.float32),
    mesh=plsc.VectorSubcoreMesh(
        core_axis_name="c", subcore_axis_name="s", num_cores=_NC, num_subcores=_NS
    ),
    scratch_types=[
        pltpu.VMEM((_ROWS,), jnp.int32),
        pltpu.VMEM((_SLAB, _ROWS), jnp.float32),
        pltpu.VMEM((_SLAB, _ROWS), jnp.float32),
        pltpu.VMEM((16,), jnp.float32),
        pltpu.SemaphoreType.DMA,
        pltpu.SemaphoreType.DMA,
    ],
)


def kernel(lsm, target):
    lsmt = lsm.T
    tgt = target.astype(jnp.int32)
    sc_partials = _sc_call(lsmt, tgt)
    tc_total = _tc_call(lsmt, tgt.reshape(1, _ROWS))
    return -(jnp.sum(sc_partials) + tc_total[0, 0]) / _ROWS
